# baseline (speedup 1.0000x reference)
import functools

import numpy as _np

import jax
import jax.numpy as jnp
from jax import lax
from jax.experimental import pallas as pl
from jax.experimental.pallas import tpu as pltpu
from jax.experimental.pallas import tpu_sc as plsc

_N = 10000
_NP = 10240
_F = 16
_CH = 128
_NC = 2
_NS = 16
_NW = _NC * _NS
_RPT = _NP // _NS
_HR = _RPT // _NC

_LN2 = 0.6931471805599453

_mesh = plsc.VectorSubcoreMesh(core_axis_name="c", subcore_axis_name="s")
_params = pltpu.CompilerParams(use_tc_tiling_on_sc=False)


def _rsqrt_s(d):
    d = jnp.maximum(d, 1.0)
    bits = lax.bitcast_convert_type(d, jnp.int32)
    y = lax.bitcast_convert_type(jnp.int32(0x5F3759DF) - (bits >> 1),
                                 jnp.float32)
    for _ in range(2):
        y = y * (1.5 - 0.5 * d * y * y)
    return y


def _ln_s(v):
    bits = lax.bitcast_convert_type(v, jnp.int32)
    e = ((bits >> 23) - 127).astype(jnp.float32)
    m = lax.bitcast_convert_type((bits & 0x7FFFFF) | 0x3F800000, jnp.float32)
    w = m + 1.0
    r = jnp.float32(0.4)
    for _ in range(4):
        r = r * (2.0 - w * r)
    z = (m - 1.0) * r
    z2 = z * z
    p = 1.0 + z2 * (1.0 / 3.0 + z2 * (0.2 + z2 * (1.0 / 7.0)))
    return _LN2 * e + 2.0 * z * p


_NB = 4


def _agg_loop(tbl, acc, sidx, didx, bufs, gsems, ssems, cpt):
    for u in range(_NB):
        pltpu.async_copy(tbl.at[sidx.at[u]], bufs[u], gsems[u])

    def body(t, carry):
        b = t * _NB
        for u in range(_NB):
            j = b + u
            pltpu.make_async_copy(tbl.at[sidx.at[j]], bufs[u],
                                  gsems[u]).wait()
            pltpu.async_copy(bufs[u], acc.at[didx.at[j]], ssems[u], add=True)
        for u in range(_NB):
            j = b + u
            pltpu.make_async_copy(bufs[u], acc.at[didx.at[j]],
                                  ssems[u]).wait()

            @pl.when(j + _NB < cpt)
            def _():
                pltpu.async_copy(tbl.at[sidx.at[j + _NB]], bufs[u], gsems[u])

        return carry

    lax.fori_loop(0, cpt // _NB, body, 0)


def _h1_body(x, w1, o):
    h = jnp.dot(x[...], w1[...], preferred_element_type=jnp.float32)
    o[0:_N, :] = h
    o[_N:_NP, :] = jnp.zeros((_NP - _N, _F), jnp.float32)


def _make_deg(cpt):
    @functools.partial(
        pl.kernel,
        mesh=_mesh,
        out_type=jax.ShapeDtypeStruct((_NC, _NP), jnp.float32),
        scratch_types=[
            pltpu.VMEM((cpt, _CH), jnp.int32),
            pltpu.VMEM((_CH,), jnp.float32),
            pltpu.VMEM((_RPT,), jnp.float32),
            pltpu.VMEM_SHARED((_NP,), jnp.float32),
            pltpu.SemaphoreType.DMA,
        ],
        compiler_params=_params,
    )
    def deg(dst3, out, didx, ones_v, ibuf, acc, sems):
        c = lax.axis_index("c")
        s = lax.axis_index("s")
        wid = c * _NS + s
        fill = jnp.where(c == 0, 1.0, 0.0)

        def frow(i, carry):
            ones_v[pl.ds(i * _F, _F)] = jnp.ones((_F,), jnp.float32)
            return carry

        lax.fori_loop(0, _CH // _F, frow, 0)

        def irow(i, carry):
            ibuf[pl.ds(i * _F, _F)] = jnp.full((_F,), fill, jnp.float32)
            return carry

        lax.fori_loop(0, _RPT // _F, irow, 0)
        pltpu.sync_copy(ibuf, acc.at[pl.ds(s * _RPT, _RPT)])
        pltpu.sync_copy(dst3.at[wid], didx)
        plsc.subcore_barrier()

        pltpu.async_copy(ones_v, acc.at[didx.at[0]], sems, add=True)

        def chunk(j, carry):
            pltpu.async_copy(ones_v, acc.at[didx.at[j]], sems, add=True)
            pltpu.make_async_copy(ones_v, acc.at[didx.at[j - 1]], sems).wait()
            return carry

        lax.fori_loop(1, cpt, chunk, 0)
        pltpu.make_async_copy(ones_v, acc.at[didx.at[cpt - 1]], sems).wait()
        plsc.subcore_barrier()
        pltpu.sync_copy(acc.at[pl.ds(s * _RPT, _RPT)],
                        out.at[c, pl.ds(s * _RPT, _RPT)])

    return deg


def _make_l1(cpt):
    @functools.partial(
        pl.kernel,
        mesh=_mesh,
        out_type=jax.ShapeDtypeStruct((_NC, _NP, _F), jnp.float32),
        scratch_types=[
            pltpu.VMEM((cpt, _CH), jnp.int32),
            pltpu.VMEM((cpt, _CH), jnp.int32),
            [pltpu.VMEM((_CH, _F), jnp.float32)] * _NB,
            pltpu.VMEM((_RPT, _F), jnp.float32),
            pltpu.VMEM((_RPT, _F), jnp.float32),
            pltpu.VMEM((_RPT,), jnp.float32),
            pltpu.VMEM((_RPT,), jnp.float32),
            pltpu.VMEM_SHARED((_NP, _F), jnp.float32),
            pltpu.VMEM_SHARED((_NP, _F), jnp.float32),
            [pltpu.SemaphoreType.DMA] * _NB,
            [pltpu.SemaphoreType.DMA] * _NB,
        ],
        compiler_params=_params,
    )
    def l1(h1, degp, src3, dst3, out, sidx, didx, bufs, hbuf,
           zbuf, d0, d1, tbl, acc, gsems, ssems):
        c = lax.axis_index("c")
        s = lax.axis_index("s")
        wid = c * _NS + s
        rb = s * _RPT
        pltpu.sync_copy(degp.at[0, pl.ds(rb, _RPT)], d0)
        pltpu.sync_copy(degp.at[1, pl.ds(rb, _RPT)], d1)
        pltpu.sync_copy(h1.at[pl.ds(rb, _RPT)], hbuf)
        pltpu.sync_copy(src3.at[wid], sidx)
        pltpu.sync_copy(dst3.at[wid], didx)

        def rowblk(i, carry):
            b = i * _F
            dvec = d0[pl.ds(b, _F)] + d1[pl.ds(b, _F)]
            for k in range(_F):
                hbuf[b + k, :] = hbuf[b + k, :] * _rsqrt_s(dvec[k])
            return carry

        lax.fori_loop(0, _RPT // _F, rowblk, 0)

        def zrow(i, carry):
            zbuf[i, :] = jnp.zeros((_F,), jnp.float32)
            return carry

        lax.fori_loop(0, _RPT, zrow, 0)
        pltpu.sync_copy(hbuf, tbl.at[pl.ds(rb, _RPT)])

        @pl.when(c == 0)
        def _():
            pltpu.sync_copy(hbuf, acc.at[pl.ds(rb, _RPT)])

        @pl.when(c != 0)
        def _():
            pltpu.sync_copy(zbuf, acc.at[pl.ds(rb, _RPT)])

        plsc.subcore_barrier()
        _agg_loop(tbl, acc, sidx, didx, bufs, gsems, ssems, cpt)
        plsc.subcore_barrier()
        pltpu.sync_copy(acc.at[pl.ds(rb, _RPT)], out.at[c, pl.ds(rb, _RPT)])

    return l1


def _make_l2(cpt):
    @functools.partial(
        pl.kernel,
        mesh=_mesh,
        out_type=jax.ShapeDtypeStruct((_NC, _NP, _F), jnp.float32),
        scratch_types=[
            pltpu.VMEM((cpt, _CH), jnp.int32),
            pltpu.VMEM((cpt, _CH), jnp.int32),
            [pltpu.VMEM((_CH, _F), jnp.float32)] * _NB,
            pltpu.VMEM((_RPT, _F), jnp.float32),
            pltpu.VMEM((_RPT, _F), jnp.float32),
            pltpu.VMEM((_RPT, _F), jnp.float32),
            pltpu.VMEM((_RPT, _F), jnp.float32),
            pltpu.VMEM((_RPT,), jnp.float32),
            pltpu.VMEM((_RPT,), jnp.float32),
            pltpu.VMEM((_F, _F), jnp.float32),
            pltpu.VMEM((_F,), jnp.float32),
            pltpu.VMEM_SHARED((_NP, _F), jnp.float32),
            pltpu.VMEM_SHARED((_NP, _F), jnp.float32),
            [pltpu.SemaphoreType.DMA] * _NB,
            [pltpu.SemaphoreType.DMA] * _NB,
        ],
        compiler_params=_params,
    )
    def l2(qp, degp, b1h, w2h, src3, dst3, out, sidx, didx, bufs,
           qb, q1b, g2b, zbuf, d0, d1, w2v, b1v, tbl, acc, gsems, ssems):
        c = lax.axis_index("c")
        s = lax.axis_index("s")
        wid = c * _NS + s
        rb = s * _RPT
        pltpu.sync_copy(degp.at[0, pl.ds(rb, _RPT)], d0)
        pltpu.sync_copy(degp.at[1, pl.ds(rb, _RPT)], d1)
        pltpu.sync_copy(qp.at[0, pl.ds(rb, _RPT)], qb)
        pltpu.sync_copy(qp.at[1, pl.ds(rb, _RPT)], q1b)
        pltpu.sync_copy(w2h, w2v)
        pltpu.sync_copy(b1h, b1v)
        pltpu.sync_copy(src3.at[wid], sidx)
        pltpu.sync_copy(dst3.at[wid], didx)

        b1vec = b1v[:]
        w2rows = [w2v[k, :] for k in range(_F)]

        def rowblk(i, carry):
            b = i * _F
            dvec = d0[pl.ds(b, _F)] + d1[pl.ds(b, _F)]
            for k in range(_F):
                y = _rsqrt_s(dvec[k])
                o1 = jnp.maximum(
                    (qb[b + k, :] + q1b[b + k, :]) * y + b1vec, 0.0)
                h2 = o1[0] * w2rows[0]
                for m in range(1, _F):
                    h2 = h2 + o1[m] * w2rows[m]
                g2b[b + k, :] = h2 * y
            return carry

        lax.fori_loop(0, _RPT // _F, rowblk, 0)

        def zrow(i, carry):
            zbuf[i, :] = jnp.zeros((_F,), jnp.float32)
            return carry

        lax.fori_loop(0, _RPT, zrow, 0)
        pltpu.sync_copy(g2b, tbl.at[pl.ds(rb, _RPT)])

        @pl.when(c == 0)
        def _():
            pltpu.sync_copy(g2b, acc.at[pl.ds(rb, _RPT)])

        @pl.when(c != 0)
        def _():
            pltpu.sync_copy(zbuf, acc.at[pl.ds(rb, _RPT)])

        plsc.subcore_barrier()
        _agg_loop(tbl, acc, sidx, didx, bufs, gsems, ssems, cpt)
        plsc.subcore_barrier()
        pltpu.sync_copy(acc.at[pl.ds(rb, _RPT)], out.at[c, pl.ds(rb, _RPT)])

    return l2


def _make_fin():
    @functools.partial(
        pl.kernel,
        mesh=_mesh,
        out_type=jax.ShapeDtypeStruct((_N, _F), jnp.float32),
        scratch_types=[
            pltpu.VMEM((_HR, _F), jnp.float32),
            pltpu.VMEM((_HR, _F), jnp.float32),
            pltpu.VMEM((_HR,), jnp.float32),
            pltpu.VMEM((_HR,), jnp.float32),
            pltpu.VMEM((_F,), jnp.float32),
        ],
        compiler_params=_params,
    )
    def fin(rp, degp, b2h, out, obuf, r1b, d0, d1, b2v):
        c = lax.axis_index("c")
        s = lax.axis_index("s")
        wid = c * _NS + s
        rb = s * _RPT + c * _HR
        pltpu.sync_copy(rp.at[0, pl.ds(rb, _HR)], obuf)
        pltpu.sync_copy(rp.at[1, pl.ds(rb, _HR)], r1b)
        pltpu.sync_copy(degp.at[0, pl.ds(rb, _HR)], d0)
        pltpu.sync_copy(degp.at[1, pl.ds(rb, _HR)], d1)
        pltpu.sync_copy(b2h, b2v)
        b2vec = b2v[:]

        def rowblk(i, carry):
            b = i * _F
            dvec = d0[pl.ds(b, _F)] + d1[pl.ds(b, _F)]
            for k in range(_F):
                obuf[b + k, :] = (obuf[b + k, :] + r1b[b + k, :]) \
                    * _rsqrt_s(dvec[k]) + b2vec
            return carry

        lax.fori_loop(0, _HR // _F, rowblk, 0)

        def blk(i, carry):
            b = i * _F
            for k in range(_F):
                row = obuf[b + k, :]
                e = jnp.exp(row)
                s = e[0]
                for m in range(1, _F):
                    s = s + e[m]
                obuf[b + k, :] = row - _ln_s(s)
            return carry

        lax.fori_loop(0, _HR // _F, blk, 0)

        @pl.when(wid != _NW - 1)
        def _():
            pltpu.sync_copy(obuf, out.at[pl.ds(rb, _HR)])

        @pl.when(wid == _NW - 1)
        def _():
            pltpu.sync_copy(obuf.at[pl.ds(0, _N - (_NP - _HR))],
                            out.at[pl.ds(_NP - _HR, _N - (_NP - _HR))])

    return fin


def kernel(x, edge_index, W1, b1, W2, b2):
    e = edge_index.shape[1]
    e2p = -(-e // (2 * _NW * _CH)) * (2 * _NW * _CH)
    pad = e2p - e
    cpt = e2p // (_NW * _CH)

    padblk = jnp.asarray(
        _N + _np.arange(pad, dtype=_np.int32) % (_NP - _N), dtype=jnp.int32)
    ei = jnp.concatenate(
        [edge_index, jnp.broadcast_to(padblk, (2, pad))], axis=1
    ).reshape(2, _NW, cpt, _CH)
    src3 = ei[0]
    dst3 = ei[1]

    h1 = pl.pallas_call(
        _h1_body,
        out_shape=jax.ShapeDtypeStruct((_NP, _F), jnp.float32),
    )(x, W1)

    degp = _make_deg(cpt)(dst3)
    q = _make_l1(cpt)(h1, degp, src3, dst3)
    r = _make_l2(cpt)(q, degp, b1, W2, src3, dst3)
    return _make_fin()(r, degp, b2)


# speedup vs baseline: 1.0903x; 1.0903x over previous
import functools

import numpy as _np

import jax
import jax.numpy as jnp
from jax import lax
from jax.experimental import pallas as pl
from jax.experimental.pallas import tpu as pltpu
from jax.experimental.pallas import tpu_sc as plsc

_N = 10000
_NP = 10240
_F = 16
_CH = 128
_NC = 2
_NS = 16
_NW = _NC * _NS
_RPT = _NP // _NS
_HR = _RPT // _NC

_LN2 = 0.6931471805599453

_mesh = plsc.VectorSubcoreMesh(core_axis_name="c", subcore_axis_name="s")
_params = pltpu.CompilerParams(use_tc_tiling_on_sc=False)


def _rsqrt_s(d):
    d = jnp.maximum(d, 1.0)
    bits = lax.bitcast_convert_type(d, jnp.int32)
    y = lax.bitcast_convert_type(jnp.int32(0x5F3759DF) - (bits >> 1),
                                 jnp.float32)
    for _ in range(2):
        y = y * (1.5 - 0.5 * d * y * y)
    return y


def _ln_s(v):
    bits = lax.bitcast_convert_type(v, jnp.int32)
    e = ((bits >> 23) - 127).astype(jnp.float32)
    m = lax.bitcast_convert_type((bits & 0x7FFFFF) | 0x3F800000, jnp.float32)
    w = m + 1.0
    r = jnp.float32(0.4)
    for _ in range(4):
        r = r * (2.0 - w * r)
    z = (m - 1.0) * r
    z2 = z * z
    p = 1.0 + z2 * (1.0 / 3.0 + z2 * (0.2 + z2 * (1.0 / 7.0)))
    return _LN2 * e + 2.0 * z * p


_G = 2


def _agg_loop(tbl, acc, sidx, didx, rows0, rows1, semg, nch):
    cph = nch // 2

    def si(j):
        return sidx.at[j]

    def di(j):
        return didx.at[j]

    pltpu.async_copy(tbl.at[si(0)], rows0, semg)

    def body(t, carry):
        j0 = 2 * t
        j1 = j0 + 1
        pltpu.make_async_copy(tbl.at[si(j0)], rows0, semg).wait()
        pltpu.async_copy(tbl.at[si(j1)], rows1, semg)
        pltpu.sync_copy(rows0, acc.at[di(j0)], add=True)
        pltpu.make_async_copy(tbl.at[si(j1)], rows1, semg).wait()

        @pl.when(t + 1 < cph)
        def _():
            pltpu.async_copy(tbl.at[si(j0 + 2)], rows0, semg)

        pltpu.sync_copy(rows1, acc.at[di(j1)], add=True)
        return carry

    lax.fori_loop(0, cph, body, 0)


def _h1_body(x, w1, o):
    h = jnp.dot(x[...], w1[...], preferred_element_type=jnp.float32)
    o[0:_N, :] = h
    o[_N:_NP, :] = jnp.zeros((_NP - _N, _F), jnp.float32)


def _make_deg(cpt):
    @functools.partial(
        pl.kernel,
        mesh=_mesh,
        out_type=jax.ShapeDtypeStruct((_NC, _NP), jnp.float32),
        scratch_types=[
            pltpu.VMEM((cpt // _G, _G * _CH), jnp.int32),
            pltpu.VMEM((_G * _CH,), jnp.float32),
            pltpu.VMEM((_RPT,), jnp.float32),
            pltpu.VMEM_SHARED((_NP,), jnp.float32),
            pltpu.SemaphoreType.DMA,
        ],
        compiler_params=_params,
    )
    def deg(dst3, out, didx, ones_v, ibuf, acc, sems):
        c = lax.axis_index("c")
        s = lax.axis_index("s")
        wid = c * _NS + s
        fill = jnp.where(c == 0, 1.0, 0.0)

        def frow(i, carry):
            ones_v[pl.ds(i * _F, _F)] = jnp.ones((_F,), jnp.float32)
            return carry

        lax.fori_loop(0, _G * _CH // _F, frow, 0)

        def irow(i, carry):
            ibuf[pl.ds(i * _F, _F)] = jnp.full((_F,), fill, jnp.float32)
            return carry

        lax.fori_loop(0, _RPT // _F, irow, 0)
        pltpu.sync_copy(ibuf, acc.at[pl.ds(s * _RPT, _RPT)])
        pltpu.sync_copy(dst3.at[wid], didx)
        plsc.subcore_barrier()

        pltpu.async_copy(ones_v, acc.at[didx.at[0]], sems, add=True)

        def chunk(j, carry):
            pltpu.async_copy(ones_v, acc.at[didx.at[j]], sems, add=True)
            pltpu.make_async_copy(ones_v, acc.at[didx.at[j - 1]], sems).wait()
            return carry

        lax.fori_loop(1, cpt // _G, chunk, 0)
        pltpu.make_async_copy(ones_v, acc.at[didx.at[cpt // _G - 1]],
                              sems).wait()
        plsc.subcore_barrier()
        pltpu.sync_copy(acc.at[pl.ds(s * _RPT, _RPT)],
                        out.at[c, pl.ds(s * _RPT, _RPT)])

    return deg


def _make_l1(cpt):
    @functools.partial(
        pl.kernel,
        mesh=_mesh,
        out_type=jax.ShapeDtypeStruct((_NC, _NP, _F), jnp.float32),
        scratch_types=[
            pltpu.VMEM((cpt // _G, _G * _CH), jnp.int32),
            pltpu.VMEM((cpt // _G, _G * _CH), jnp.int32),
            pltpu.VMEM((_G * _CH, _F), jnp.float32),
            pltpu.VMEM((_G * _CH, _F), jnp.float32),
            pltpu.VMEM((_RPT, _F), jnp.float32),
            pltpu.VMEM((_RPT, _F), jnp.float32),
            pltpu.VMEM((_RPT,), jnp.float32),
            pltpu.VMEM((_RPT,), jnp.float32),
            pltpu.VMEM_SHARED((_NP, _F), jnp.float32),
            pltpu.VMEM_SHARED((_NP, _F), jnp.float32),
            pltpu.SemaphoreType.DMA,
        ],
        compiler_params=_params,
    )
    def l1(h1, degp, src3, dst3, out, sidx, didx, rows0, rows1, hbuf,
           zbuf, d0, d1, tbl, acc, semg):
        c = lax.axis_index("c")
        s = lax.axis_index("s")
        wid = c * _NS + s
        rb = s * _RPT
        pltpu.sync_copy(degp.at[0, pl.ds(rb, _RPT)], d0)
        pltpu.sync_copy(degp.at[1, pl.ds(rb, _RPT)], d1)
        pltpu.sync_copy(h1.at[pl.ds(rb, _RPT)], hbuf)
        pltpu.sync_copy(src3.at[wid], sidx)
        pltpu.sync_copy(dst3.at[wid], didx)

        def rowblk(i, carry):
            b = i * _F
            dvec = d0[pl.ds(b, _F)] + d1[pl.ds(b, _F)]
            for k in range(_F):
                hbuf[b + k, :] = hbuf[b + k, :] * _rsqrt_s(dvec[k])
            return carry

        lax.fori_loop(0, _RPT // _F, rowblk, 0)

        def zrow(i, carry):
            zbuf[i, :] = jnp.zeros((_F,), jnp.float32)
            return carry

        lax.fori_loop(0, _RPT, zrow, 0)
        pltpu.sync_copy(hbuf, tbl.at[pl.ds(rb, _RPT)])

        @pl.when(c == 0)
        def _():
            pltpu.sync_copy(hbuf, acc.at[pl.ds(rb, _RPT)])

        @pl.when(c != 0)
        def _():
            pltpu.sync_copy(zbuf, acc.at[pl.ds(rb, _RPT)])

        plsc.subcore_barrier()
        _agg_loop(tbl, acc, sidx, didx, rows0, rows1, semg, cpt // _G)
        plsc.subcore_barrier()
        pltpu.sync_copy(acc.at[pl.ds(rb, _RPT)], out.at[c, pl.ds(rb, _RPT)])

    return l1


def _make_l2(cpt):
    @functools.partial(
        pl.kernel,
        mesh=_mesh,
        out_type=jax.ShapeDtypeStruct((_NC, _NP, _F), jnp.float32),
        scratch_types=[
            pltpu.VMEM((cpt // _G, _G * _CH), jnp.int32),
            pltpu.VMEM((cpt // _G, _G * _CH), jnp.int32),
            pltpu.VMEM((_G * _CH, _F), jnp.float32),
            pltpu.VMEM((_G * _CH, _F), jnp.float32),
            pltpu.VMEM((_RPT, _F), jnp.float32),
            pltpu.VMEM((_RPT, _F), jnp.float32),
            pltpu.VMEM((_RPT, _F), jnp.float32),
            pltpu.VMEM((_RPT, _F), jnp.float32),
            pltpu.VMEM((_RPT,), jnp.float32),
            pltpu.VMEM((_RPT,), jnp.float32),
            pltpu.VMEM((_F, _F), jnp.float32),
            pltpu.VMEM((_F,), jnp.float32),
            pltpu.VMEM_SHARED((_NP, _F), jnp.float32),
            pltpu.VMEM_SHARED((_NP, _F), jnp.float32),
            pltpu.SemaphoreType.DMA,
        ],
        compiler_params=_params,
    )
    def l2(qp, degp, b1h, w2h, src3, dst3, out, sidx, didx, rows0, rows1,
           qb, q1b, g2b, zbuf, d0, d1, w2v, b1v, tbl, acc, semg):
        c = lax.axis_index("c")
        s = lax.axis_index("s")
        wid = c * _NS + s
        rb = s * _RPT
        pltpu.sync_copy(degp.at[0, pl.ds(rb, _RPT)], d0)
        pltpu.sync_copy(degp.at[1, pl.ds(rb, _RPT)], d1)
        pltpu.sync_copy(qp.at[0, pl.ds(rb, _RPT)], qb)
        pltpu.sync_copy(qp.at[1, pl.ds(rb, _RPT)], q1b)
        pltpu.sync_copy(w2h, w2v)
        pltpu.sync_copy(b1h, b1v)
        pltpu.sync_copy(src3.at[wid], sidx)
        pltpu.sync_copy(dst3.at[wid], didx)

        b1vec = b1v[:]
        w2rows = [w2v[k, :] for k in range(_F)]

        def rowblk(i, carry):
            b = i * _F
            dvec = d0[pl.ds(b, _F)] + d1[pl.ds(b, _F)]
            for k in range(_F):
                y = _rsqrt_s(dvec[k])
                o1 = jnp.maximum(
                    (qb[b + k, :] + q1b[b + k, :]) * y + b1vec, 0.0)
                h2 = o1[0] * w2rows[0]
                for m in range(1, _F):
                    h2 = h2 + o1[m] * w2rows[m]
                g2b[b + k, :] = h2 * y
            return carry

        lax.fori_loop(0, _RPT // _F, rowblk, 0)

        def zrow(i, carry):
            zbuf[i, :] = jnp.zeros((_F,), jnp.float32)
            return carry

        lax.fori_loop(0, _RPT, zrow, 0)
        pltpu.sync_copy(g2b, tbl.at[pl.ds(rb, _RPT)])

        @pl.when(c == 0)
        def _():
            pltpu.sync_copy(g2b, acc.at[pl.ds(rb, _RPT)])

        @pl.when(c != 0)
        def _():
            pltpu.sync_copy(zbuf, acc.at[pl.ds(rb, _RPT)])

        plsc.subcore_barrier()
        _agg_loop(tbl, acc, sidx, didx, rows0, rows1, semg, cpt // _G)
        plsc.subcore_barrier()
        pltpu.sync_copy(acc.at[pl.ds(rb, _RPT)], out.at[c, pl.ds(rb, _RPT)])

    return l2


def _make_fin():
    @functools.partial(
        pl.kernel,
        mesh=_mesh,
        out_type=jax.ShapeDtypeStruct((_N, _F), jnp.float32),
        scratch_types=[
            pltpu.VMEM((_HR, _F), jnp.float32),
            pltpu.VMEM((_HR, _F), jnp.float32),
            pltpu.VMEM((_HR,), jnp.float32),
            pltpu.VMEM((_HR,), jnp.float32),
            pltpu.VMEM((_F,), jnp.float32),
        ],
        compiler_params=_params,
    )
    def fin(rp, degp, b2h, out, obuf, r1b, d0, d1, b2v):
        c = lax.axis_index("c")
        s = lax.axis_index("s")
        wid = c * _NS + s
        rb = s * _RPT + c * _HR
        pltpu.sync_copy(rp.at[0, pl.ds(rb, _HR)], obuf)
        pltpu.sync_copy(rp.at[1, pl.ds(rb, _HR)], r1b)
        pltpu.sync_copy(degp.at[0, pl.ds(rb, _HR)], d0)
        pltpu.sync_copy(degp.at[1, pl.ds(rb, _HR)], d1)
        pltpu.sync_copy(b2h, b2v)
        b2vec = b2v[:]

        def rowblk(i, carry):
            b = i * _F
            dvec = d0[pl.ds(b, _F)] + d1[pl.ds(b, _F)]
            for k in range(_F):
                obuf[b + k, :] = (obuf[b + k, :] + r1b[b + k, :]) \
                    * _rsqrt_s(dvec[k]) + b2vec
            return carry

        lax.fori_loop(0, _HR // _F, rowblk, 0)

        def blk(i, carry):
            b = i * _F
            for k in range(_F):
                row = obuf[b + k, :]
                e = jnp.exp(row)
                s = e[0]
                for m in range(1, _F):
                    s = s + e[m]
                obuf[b + k, :] = row - _ln_s(s)
            return carry

        lax.fori_loop(0, _HR // _F, blk, 0)

        @pl.when(wid != _NW - 1)
        def _():
            pltpu.sync_copy(obuf, out.at[pl.ds(rb, _HR)])

        @pl.when(wid == _NW - 1)
        def _():
            pltpu.sync_copy(obuf.at[pl.ds(0, _N - (_NP - _HR))],
                            out.at[pl.ds(_NP - _HR, _N - (_NP - _HR))])

    return fin


def kernel(x, edge_index, W1, b1, W2, b2):
    e = edge_index.shape[1]
    e2p = -(-e // (2 * _NW * _CH)) * (2 * _NW * _CH)
    pad = e2p - e
    cpt = e2p // (_NW * _CH)

    padblk = jnp.asarray(
        _N + _np.arange(pad, dtype=_np.int32) % (_NP - _N), dtype=jnp.int32)
    ei = jnp.concatenate(
        [edge_index, jnp.broadcast_to(padblk, (2, pad))], axis=1
    ).reshape(2, _NW, cpt // _G, _G * _CH)
    src3 = ei[0]
    dst3 = ei[1]

    h1 = pl.pallas_call(
        _h1_body,
        out_shape=jax.ShapeDtypeStruct((_NP, _F), jnp.float32),
    )(x, W1)

    degp = _make_deg(cpt)(dst3)
    q = _make_l1(cpt)(h1, degp, src3, dst3)
    r = _make_l2(cpt)(q, degp, b1, W2, src3, dst3)
    return _make_fin()(r, degp, b2)
